# baseline (speedup 1.0000x reference)
import jax
import jax.numpy as jnp
from jax.experimental import pallas as pl
from jax.experimental.pallas import tpu as pltpu

_VMEM_LIMIT = 60 * 1024 * 1024
_N_CORES = 2


def _effective_weight_t(w_hat, m_hat):
    w = jnp.tanh(w_hat) * jax.nn.sigmoid(m_hat)
    return w.T.astype(jnp.bfloat16)


def _make_manual_kernel(chunk, n_steps, k, n, halves, o_slots):
    rows_per_core = chunk * n_steps
    half = chunk // halves
    n_units = n_steps * halves

    def _kernel(x_hbm, wh_ref, mh_ref, y_hbm,
                x_buf, o_buf, wt_v, in_sem, out_sem):
        base = pl.program_id(0) * rows_per_core

        def dma_in(slot, step):
            pltpu.make_async_copy(
                x_hbm.at[pl.ds(base + step * chunk, chunk), :],
                x_buf.at[slot], in_sem.at[slot]).start()

        def wait_in(slot):
            pltpu.make_async_copy(
                x_hbm.at[pl.ds(0, chunk), :],
                x_buf.at[slot], in_sem.at[slot]).wait()

        def dma_out(slot, unit):
            pltpu.make_async_copy(
                o_buf.at[slot],
                y_hbm.at[pl.ds(base + unit * half, half), :],
                out_sem.at[slot]).start()

        def wait_out(slot):
            pltpu.make_async_copy(
                o_buf.at[slot],
                y_hbm.at[pl.ds(0, half), :],
                out_sem.at[slot]).wait()

        for s in range(min(3, n_steps)):
            dma_in(s, s)
        wt_v[...] = _effective_weight_t(wh_ref[...], mh_ref[...])

        for step in range(n_steps):
            cur = step % 4
            if step + 3 < n_steps:
                dma_in((step + 3) % 4, step + 3)
            wait_in(cur)
            for h in range(halves):
                u = step * halves + h
                us = u % o_slots
                if u >= o_slots:
                    wait_out(us)
                o_buf[us] = jnp.dot(
                    x_buf[cur, pl.ds(h * half, half), :].astype(jnp.bfloat16),
                    wt_v[...], preferred_element_type=jnp.float32)
                dma_out(us, u)
        for t in range(max(0, n_units - o_slots), n_units):
            wait_out(t % o_slots)

    return _kernel


def _fused_emitter_kernel(x_ref, w_hat_ref, m_hat_ref, o_ref):
    wt = _effective_weight_t(w_hat_ref[...], m_hat_ref[...])
    o_ref[...] = jnp.dot(
        x_ref[...].astype(jnp.bfloat16), wt,
        preferred_element_type=jnp.float32).astype(o_ref.dtype)


def _largest_divisor_tile(size, cap, align):
    best = align
    t = align
    while t <= min(cap, size):
        if size % t == 0:
            best = t
        t += align
    return best


def kernel(x, w_hat, m_hat):
    batch, in_dim = x.shape
    out_dim = w_hat.shape[0]
    out_dtype = x.dtype

    def _up(v, m):
        return (v + m - 1) // m * m

    B = _up(batch, 8)
    K = _up(in_dim, 128)
    N = _up(out_dim, 128)
    if (B, K) != (batch, in_dim):
        x = jnp.pad(x, ((0, B - batch), (0, K - in_dim)))
    if (N, K) != (out_dim, in_dim):
        pad_w = ((0, N - out_dim), (0, K - in_dim))
        w_hat = jnp.pad(w_hat, pad_w)
        m_hat = jnp.pad(m_hat, pad_w)

    rows_per_core = B // _N_CORES
    chunk = _largest_divisor_tile(rows_per_core, 2048, 8) \
        if rows_per_core % 8 == 0 else 0
    vmem_ok = chunk and (4 * chunk * (K + N) * 4 +
                         (2 * 4 + 2) * N * K) < _VMEM_LIMIT

    if B % _N_CORES == 0 and chunk and vmem_ok:
        n_steps = rows_per_core // chunk
        halves = 2 if chunk % 16 == 0 else 1
        o_slots = min(n_steps * halves, 8)
        y = pl.pallas_call(
            _make_manual_kernel(chunk, n_steps, K, N, halves, o_slots),
            grid=(_N_CORES,),
            in_specs=[
                pl.BlockSpec(memory_space=pl.ANY),
                pl.BlockSpec((N, K), lambda i: (0, 0)),
                pl.BlockSpec((N, K), lambda i: (0, 0)),
            ],
            out_specs=pl.BlockSpec(memory_space=pl.ANY),
            out_shape=jax.ShapeDtypeStruct((B, N), jnp.float32),
            scratch_shapes=[
                pltpu.VMEM((4, chunk, K), jnp.float32),
                pltpu.VMEM((o_slots, chunk // halves, N), jnp.float32),
                pltpu.VMEM((K, N), jnp.bfloat16),
                pltpu.SemaphoreType.DMA((4,)),
                pltpu.SemaphoreType.DMA((o_slots,)),
            ],
            compiler_params=pltpu.CompilerParams(
                dimension_semantics=("parallel",),
                vmem_limit_bytes=_VMEM_LIMIT,
            ),
        )(x, w_hat, m_hat)
    else:
        tm = _largest_divisor_tile(B, 4096, 8)
        y = pl.pallas_call(
            _fused_emitter_kernel,
            grid=(B // tm,),
            in_specs=[
                pl.BlockSpec((tm, K), lambda i: (i, 0)),
                pl.BlockSpec((N, K), lambda i: (0, 0)),
                pl.BlockSpec((N, K), lambda i: (0, 0)),
            ],
            out_specs=pl.BlockSpec((tm, N), lambda i: (i, 0)),
            out_shape=jax.ShapeDtypeStruct((B, N), jnp.float32),
            compiler_params=pltpu.CompilerParams(
                dimension_semantics=("parallel",),
                vmem_limit_bytes=_VMEM_LIMIT,
            ),
        )(x, w_hat, m_hat)

    if out_dtype != jnp.float32:
        y = y.astype(out_dtype)
    if (B, N) != (batch, out_dim):
        y = y[:batch, :out_dim]
    return y


# speedup vs baseline: 1.0144x; 1.0144x over previous
import jax
import jax.numpy as jnp
from jax.experimental import pallas as pl
from jax.experimental.pallas import tpu as pltpu

_VMEM_LIMIT = 60 * 1024 * 1024
_N_CORES = 2


def _effective_weight_t(w_hat, m_hat):
    w = jnp.tanh(w_hat) * jax.nn.sigmoid(m_hat)
    return w.T.astype(jnp.bfloat16)


def _make_manual_kernel(chunk, n_steps, k, n, halves, o_slots):
    rows_per_core = chunk * n_steps
    half = chunk // halves
    n_units = n_steps * halves

    def _kernel(x_hbm, wh_ref, mh_ref, y_hbm,
                x_buf, o_buf, wt_v, in_sem, out_sem):
        base = pl.program_id(0) * rows_per_core

        def dma_in(slot, step):
            pltpu.make_async_copy(
                x_hbm.at[pl.ds(base + step * chunk, chunk), :],
                x_buf.at[slot], in_sem.at[slot]).start()

        def wait_in(slot):
            pltpu.make_async_copy(
                x_hbm.at[pl.ds(0, chunk), :],
                x_buf.at[slot], in_sem.at[slot]).wait()

        def dma_out(slot, unit):
            pltpu.make_async_copy(
                o_buf.at[slot],
                y_hbm.at[pl.ds(base + unit * half, half), :],
                out_sem.at[slot]).start()

        def wait_out(slot):
            pltpu.make_async_copy(
                o_buf.at[slot],
                y_hbm.at[pl.ds(0, half), :],
                out_sem.at[slot]).wait()

        for s in range(min(3, n_steps)):
            dma_in(s, s)
        wt_v[...] = _effective_weight_t(wh_ref[...], mh_ref[...])

        for step in range(n_steps):
            cur = step % 4
            if step + 3 < n_steps:
                dma_in((step + 3) % 4, step + 3)
            wait_in(cur)
            for h in range(halves):
                u = step * halves + h
                us = u % o_slots
                if u >= o_slots:
                    wait_out(us)
                o_buf[us] = jnp.dot(
                    x_buf[cur, pl.ds(h * half, half), :].astype(jnp.bfloat16),
                    wt_v[...], preferred_element_type=jnp.float32)
                dma_out(us, u)
        for t in range(max(0, n_units - o_slots), n_units):
            wait_out(t % o_slots)

    return _kernel


def _fused_emitter_kernel(x_ref, w_hat_ref, m_hat_ref, o_ref):
    wt = _effective_weight_t(w_hat_ref[...], m_hat_ref[...])
    o_ref[...] = jnp.dot(
        x_ref[...].astype(jnp.bfloat16), wt,
        preferred_element_type=jnp.float32).astype(o_ref.dtype)


def _largest_divisor_tile(size, cap, align):
    best = align
    t = align
    while t <= min(cap, size):
        if size % t == 0:
            best = t
        t += align
    return best


def kernel(x, w_hat, m_hat):
    batch, in_dim = x.shape
    out_dim = w_hat.shape[0]
    out_dtype = x.dtype

    def _up(v, m):
        return (v + m - 1) // m * m

    B = _up(batch, 8)
    K = _up(in_dim, 128)
    N = _up(out_dim, 128)
    if (B, K) != (batch, in_dim):
        x = jnp.pad(x, ((0, B - batch), (0, K - in_dim)))
    if (N, K) != (out_dim, in_dim):
        pad_w = ((0, N - out_dim), (0, K - in_dim))
        w_hat = jnp.pad(w_hat, pad_w)
        m_hat = jnp.pad(m_hat, pad_w)

    rows_per_core = B // _N_CORES
    chunk = _largest_divisor_tile(rows_per_core, 2048, 8) \
        if rows_per_core % 8 == 0 else 0
    vmem_ok = chunk and (4 * chunk * (K + N) * 4 +
                         (2 * 4 + 2) * N * K) < _VMEM_LIMIT

    if B % _N_CORES == 0 and chunk and vmem_ok:
        n_steps = rows_per_core // chunk
        halves = 1
        o_slots = min(n_steps * halves, 8)
        y = pl.pallas_call(
            _make_manual_kernel(chunk, n_steps, K, N, halves, o_slots),
            grid=(_N_CORES,),
            in_specs=[
                pl.BlockSpec(memory_space=pl.ANY),
                pl.BlockSpec((N, K), lambda i: (0, 0)),
                pl.BlockSpec((N, K), lambda i: (0, 0)),
            ],
            out_specs=pl.BlockSpec(memory_space=pl.ANY),
            out_shape=jax.ShapeDtypeStruct((B, N), jnp.float32),
            scratch_shapes=[
                pltpu.VMEM((4, chunk, K), jnp.float32),
                pltpu.VMEM((o_slots, chunk // halves, N), jnp.float32),
                pltpu.VMEM((K, N), jnp.bfloat16),
                pltpu.SemaphoreType.DMA((4,)),
                pltpu.SemaphoreType.DMA((o_slots,)),
            ],
            compiler_params=pltpu.CompilerParams(
                dimension_semantics=("parallel",),
                vmem_limit_bytes=_VMEM_LIMIT,
            ),
        )(x, w_hat, m_hat)
    else:
        tm = _largest_divisor_tile(B, 4096, 8)
        y = pl.pallas_call(
            _fused_emitter_kernel,
            grid=(B // tm,),
            in_specs=[
                pl.BlockSpec((tm, K), lambda i: (i, 0)),
                pl.BlockSpec((N, K), lambda i: (0, 0)),
                pl.BlockSpec((N, K), lambda i: (0, 0)),
            ],
            out_specs=pl.BlockSpec((tm, N), lambda i: (i, 0)),
            out_shape=jax.ShapeDtypeStruct((B, N), jnp.float32),
            compiler_params=pltpu.CompilerParams(
                dimension_semantics=("parallel",),
                vmem_limit_bytes=_VMEM_LIMIT,
            ),
        )(x, w_hat, m_hat)

    if out_dtype != jnp.float32:
        y = y.astype(out_dtype)
    if (B, N) != (batch, out_dim):
        y = y[:batch, :out_dim]
    return y
